# baseline (speedup 1.0000x reference)
import jax
import jax.numpy as jnp
from jax.experimental import pallas as pl
from jax.experimental.pallas import tpu as pltpu


def _loss_body(melt_ref, melp_ref, post_ref, mmask_ref,
               pt_ref, pp_ref, et_ref, ep_ref, ldp_ref, dur_ref, tm_ref,
               out_ref):
    b = pl.program_id(0)

    @pl.when(b == 0)
    def _():
        tm = tm_ref[...]
        pe = (pp_ref[...] - pt_ref[...]) ** 2
        ee = (ep_ref[...] - et_ref[...]) ** 2
        ldt = jnp.log(dur_ref[...] + 1.0)
        de = (ldp_ref[...] - ldt) ** 2
        out_ref[0] = 0.0
        out_ref[1] = 0.0
        out_ref[2] = 0.0
        out_ref[3] = jnp.sum(pe * tm)
        out_ref[4] = jnp.sum(ee * tm)
        out_ref[5] = jnp.sum(de * tm)
        out_ref[6] = jnp.sum(tm)
        out_ref[7] = 0.0

    t = melt_ref[0]
    m = mmask_ref[0, 0]
    mm = m[:, None]
    d1 = jnp.abs(melp_ref[0] - t) * mm
    d2 = jnp.abs(post_ref[0] - t) * mm
    out_ref[0] += jnp.sum(d1)
    out_ref[1] += jnp.sum(d2)
    out_ref[2] += jnp.sum(m)


def kernel(mel_targets, pitch_targets, energy_targets, duration_targets,
           mel_predictions, postnet_mel_predictions, pitch_predictions,
           energy_predictions, log_duration_predictions, text_masks, mel_masks):
    B, T_mel, n_mels = mel_targets.shape
    T_text = pitch_targets.shape[1]

    tm = jnp.logical_not(text_masks).astype(jnp.float32)
    mm = jnp.logical_not(mel_masks).astype(jnp.float32).reshape(B, 1, T_mel)
    dur_f = duration_targets.astype(jnp.float32)

    sums = pl.pallas_call(
        _loss_body,
        grid=(B,),
        in_specs=[
            pl.BlockSpec((1, T_mel, n_mels), lambda b: (b, 0, 0)),
            pl.BlockSpec((1, T_mel, n_mels), lambda b: (b, 0, 0)),
            pl.BlockSpec((1, T_mel, n_mels), lambda b: (b, 0, 0)),
            pl.BlockSpec((1, 1, T_mel), lambda b: (b, 0, 0)),
            pl.BlockSpec((B, T_text), lambda b: (0, 0)),
            pl.BlockSpec((B, T_text), lambda b: (0, 0)),
            pl.BlockSpec((B, T_text), lambda b: (0, 0)),
            pl.BlockSpec((B, T_text), lambda b: (0, 0)),
            pl.BlockSpec((B, T_text), lambda b: (0, 0)),
            pl.BlockSpec((B, T_text), lambda b: (0, 0)),
            pl.BlockSpec((B, T_text), lambda b: (0, 0)),
        ],
        out_specs=pl.BlockSpec(memory_space=pltpu.SMEM),
        out_shape=jax.ShapeDtypeStruct((8,), jnp.float32),
    )(mel_targets, mel_predictions, postnet_mel_predictions, mm,
      pitch_targets, pitch_predictions, energy_targets, energy_predictions,
      log_duration_predictions, dur_f, tm)

    mel_num = sums[0]
    post_num = sums[1]
    mel_msum = sums[2]
    pitch_num = sums[3]
    energy_num = sums[4]
    dur_num = sums[5]
    text_msum = sums[6]

    mel_loss = mel_num / (mel_msum * n_mels)
    postnet_mel_loss = post_num / (mel_msum * n_mels)
    pitch_loss = pitch_num / text_msum
    energy_loss = energy_num / text_msum
    duration_loss = dur_num / text_msum
    total_loss = (mel_loss + postnet_mel_loss + duration_loss
                  + pitch_loss + energy_loss)
    return (total_loss, mel_loss, postnet_mel_loss, pitch_loss,
            energy_loss, duration_loss)


# speedup vs baseline: 1.1512x; 1.1512x over previous
import jax
import jax.numpy as jnp
from jax.experimental import pallas as pl
from jax.experimental.pallas import tpu as pltpu


def _loss_body(melt_ref, melp_ref, post_ref, mmask_ref,
               pt_ref, pp_ref, et_ref, ep_ref, ldp_ref, dur_ref, tm_ref,
               out_ref):
    b = pl.program_id(0)

    @pl.when(b == 0)
    def _():
        tm = tm_ref[...]
        pe = (pp_ref[...] - pt_ref[...]) ** 2
        ee = (ep_ref[...] - et_ref[...]) ** 2
        ldt = jnp.log(dur_ref[...] + 1.0)
        de = (ldp_ref[...] - ldt) ** 2
        out_ref[0] = 0.0
        out_ref[1] = 0.0
        out_ref[2] = 0.0
        out_ref[3] = jnp.sum(pe * tm)
        out_ref[4] = jnp.sum(ee * tm)
        out_ref[5] = jnp.sum(de * tm)
        out_ref[6] = jnp.sum(tm)
        out_ref[7] = 0.0

    t = melt_ref[...]
    m = mmask_ref[...][:, 0, :]
    d1 = jnp.abs(melp_ref[...] - t)
    d2 = jnp.abs(post_ref[...] - t)
    dn = (((1,), (1,)), ((0,), (0,)))
    p1 = jax.lax.dot_general(m, d1, dn, preferred_element_type=jnp.float32)
    p2 = jax.lax.dot_general(m, d2, dn, preferred_element_type=jnp.float32)
    out_ref[0] += jnp.sum(p1)
    out_ref[1] += jnp.sum(p2)
    out_ref[2] += jnp.sum(m)


def kernel(mel_targets, pitch_targets, energy_targets, duration_targets,
           mel_predictions, postnet_mel_predictions, pitch_predictions,
           energy_predictions, log_duration_predictions, text_masks, mel_masks):
    B, T_mel, n_mels = mel_targets.shape
    T_text = pitch_targets.shape[1]

    tm = jnp.logical_not(text_masks).astype(jnp.float32)
    mm = jnp.logical_not(mel_masks).astype(jnp.float32).reshape(B, 1, T_mel)
    dur_f = duration_targets.astype(jnp.float32)

    BB = 4
    sums = pl.pallas_call(
        _loss_body,
        grid=(B // BB,),
        in_specs=[
            pl.BlockSpec((BB, T_mel, n_mels), lambda b: (b, 0, 0)),
            pl.BlockSpec((BB, T_mel, n_mels), lambda b: (b, 0, 0)),
            pl.BlockSpec((BB, T_mel, n_mels), lambda b: (b, 0, 0)),
            pl.BlockSpec((BB, 1, T_mel), lambda b: (b, 0, 0)),
            pl.BlockSpec((B, T_text), lambda b: (0, 0)),
            pl.BlockSpec((B, T_text), lambda b: (0, 0)),
            pl.BlockSpec((B, T_text), lambda b: (0, 0)),
            pl.BlockSpec((B, T_text), lambda b: (0, 0)),
            pl.BlockSpec((B, T_text), lambda b: (0, 0)),
            pl.BlockSpec((B, T_text), lambda b: (0, 0)),
            pl.BlockSpec((B, T_text), lambda b: (0, 0)),
        ],
        out_specs=pl.BlockSpec(memory_space=pltpu.SMEM),
        out_shape=jax.ShapeDtypeStruct((8,), jnp.float32),
    )(mel_targets, mel_predictions, postnet_mel_predictions, mm,
      pitch_targets, pitch_predictions, energy_targets, energy_predictions,
      log_duration_predictions, dur_f, tm)

    mel_num = sums[0]
    post_num = sums[1]
    mel_msum = sums[2]
    pitch_num = sums[3]
    energy_num = sums[4]
    dur_num = sums[5]
    text_msum = sums[6]

    mel_loss = mel_num / (mel_msum * n_mels)
    postnet_mel_loss = post_num / (mel_msum * n_mels)
    pitch_loss = pitch_num / text_msum
    energy_loss = energy_num / text_msum
    duration_loss = dur_num / text_msum
    total_loss = (mel_loss + postnet_mel_loss + duration_loss
                  + pitch_loss + energy_loss)
    return (total_loss, mel_loss, postnet_mel_loss, pitch_loss,
            energy_loss, duration_loss)


# speedup vs baseline: 1.1710x; 1.0172x over previous
import jax
import jax.numpy as jnp
from jax.experimental import pallas as pl
from jax.experimental.pallas import tpu as pltpu


def _loss_body(melt_ref, melp_ref, post_ref, mmask_ref,
               pt_ref, pp_ref, et_ref, ep_ref, ldp_ref, dur_ref, tm_ref,
               out_ref):
    b = pl.program_id(0)

    @pl.when(b == 0)
    def _():
        tm = tm_ref[...]
        pe = (pp_ref[...] - pt_ref[...]) ** 2
        ee = (ep_ref[...] - et_ref[...]) ** 2
        ldt = jnp.log(dur_ref[...] + 1.0)
        de = (ldp_ref[...] - ldt) ** 2
        out_ref[0] = 0.0
        out_ref[1] = 0.0
        out_ref[2] = 0.0
        out_ref[3] = jnp.sum(pe * tm)
        out_ref[4] = jnp.sum(ee * tm)
        out_ref[5] = jnp.sum(de * tm)
        out_ref[6] = jnp.sum(tm)
        out_ref[7] = 0.0

    t = melt_ref[...]
    m = mmask_ref[...][:, 0, :]
    d1 = jnp.abs(melp_ref[...] - t)
    d2 = jnp.abs(post_ref[...] - t)
    dn = (((1,), (1,)), ((0,), (0,)))
    p1 = jax.lax.dot_general(m, d1, dn, preferred_element_type=jnp.float32)
    p2 = jax.lax.dot_general(m, d2, dn, preferred_element_type=jnp.float32)
    out_ref[0] += jnp.sum(p1)
    out_ref[1] += jnp.sum(p2)
    out_ref[2] += jnp.sum(m)


def kernel(mel_targets, pitch_targets, energy_targets, duration_targets,
           mel_predictions, postnet_mel_predictions, pitch_predictions,
           energy_predictions, log_duration_predictions, text_masks, mel_masks):
    B, T_mel, n_mels = mel_targets.shape
    T_text = pitch_targets.shape[1]

    tm = jnp.logical_not(text_masks).astype(jnp.float32)
    mm = jnp.logical_not(mel_masks).astype(jnp.float32).reshape(B, 1, T_mel)
    dur_f = duration_targets.astype(jnp.float32)

    BB = 8
    sums = pl.pallas_call(
        _loss_body,
        grid=(B // BB,),
        in_specs=[
            pl.BlockSpec((BB, T_mel, n_mels), lambda b: (b, 0, 0)),
            pl.BlockSpec((BB, T_mel, n_mels), lambda b: (b, 0, 0)),
            pl.BlockSpec((BB, T_mel, n_mels), lambda b: (b, 0, 0)),
            pl.BlockSpec((BB, 1, T_mel), lambda b: (b, 0, 0)),
            pl.BlockSpec((B, T_text), lambda b: (0, 0)),
            pl.BlockSpec((B, T_text), lambda b: (0, 0)),
            pl.BlockSpec((B, T_text), lambda b: (0, 0)),
            pl.BlockSpec((B, T_text), lambda b: (0, 0)),
            pl.BlockSpec((B, T_text), lambda b: (0, 0)),
            pl.BlockSpec((B, T_text), lambda b: (0, 0)),
            pl.BlockSpec((B, T_text), lambda b: (0, 0)),
        ],
        out_specs=pl.BlockSpec(memory_space=pltpu.SMEM),
        out_shape=jax.ShapeDtypeStruct((8,), jnp.float32),
    )(mel_targets, mel_predictions, postnet_mel_predictions, mm,
      pitch_targets, pitch_predictions, energy_targets, energy_predictions,
      log_duration_predictions, dur_f, tm)

    mel_num = sums[0]
    post_num = sums[1]
    mel_msum = sums[2]
    pitch_num = sums[3]
    energy_num = sums[4]
    dur_num = sums[5]
    text_msum = sums[6]

    mel_loss = mel_num / (mel_msum * n_mels)
    postnet_mel_loss = post_num / (mel_msum * n_mels)
    pitch_loss = pitch_num / text_msum
    energy_loss = energy_num / text_msum
    duration_loss = dur_num / text_msum
    total_loss = (mel_loss + postnet_mel_loss + duration_loss
                  + pitch_loss + energy_loss)
    return (total_loss, mel_loss, postnet_mel_loss, pitch_loss,
            energy_loss, duration_loss)
